# baseline (speedup 1.0000x reference)
import jax
import jax.numpy as jnp
from jax import lax
from jax.experimental import pallas as pl
from jax.experimental.pallas import tpu as pltpu

N_TOK = 65536
C = 1024
D = 64
BN2 = 512
NB = (N_TOK // 2) // BN2


def _codebook_kernel(mp_ref, eps_ref, phi_ref, w2_ref, p22_ref):
    mp = mp_ref[...]
    eps = eps_ref[...]
    phi_ref[...] = jnp.dot(mp, eps, preferred_element_type=jnp.float32)
    phit = lax.dot_general(eps, mp, (((0,), (1,)), ((), ())),
                           preferred_element_type=jnp.float32)
    phit2 = -2.0 * phit
    z = jnp.zeros_like(phit2)
    w2_ref[...] = jnp.concatenate(
        [jnp.concatenate([phit2, z], axis=1),
         jnp.concatenate([z, phit2], axis=1)], axis=0)
    p2 = jnp.sum(phit * phit, axis=0, keepdims=True)
    p22_ref[...] = jnp.concatenate([p2, p2], axis=1)


def _vq_kernel(x_ref, w2_ref, p22_ref, phi_ref, out_ref):
    x = x_ref[...]
    xp2 = jnp.dot(x, w2_ref[...], preferred_element_type=jnp.float32)
    d2 = xp2 + p22_ref[...]
    phi = phi_ref[...]
    halves = []
    for h in range(2):
        dh = d2[:, h * C:(h + 1) * C]
        idx = jnp.argmin(dh, axis=1, keepdims=True).astype(jnp.int32)
        ids = lax.broadcasted_iota(jnp.int32, dh.shape, 1)
        onehot = jnp.where(ids == idx, 1.0, 0.0)
        halves.append(jnp.dot(onehot, phi,
                              preferred_element_type=jnp.float32))
    out_ref[...] = jnp.concatenate(halves, axis=1)


def kernel(X, matrix_parents, epsilon):
    phi, w2, p22 = pl.pallas_call(
        _codebook_kernel,
        out_shape=[
            jax.ShapeDtypeStruct((C, D), jnp.float32),
            jax.ShapeDtypeStruct((2 * D, 2 * C), jnp.float32),
            jax.ShapeDtypeStruct((1, 2 * C), jnp.float32),
        ],
    )(matrix_parents, epsilon)

    x2 = X.reshape(N_TOK // 2, 2 * D)
    out2 = pl.pallas_call(
        _vq_kernel,
        grid=(NB,),
        in_specs=[
            pl.BlockSpec((BN2, 2 * D), lambda i: (i, 0)),
            pl.BlockSpec((2 * D, 2 * C), lambda i: (0, 0)),
            pl.BlockSpec((1, 2 * C), lambda i: (0, 0)),
            pl.BlockSpec((C, D), lambda i: (0, 0)),
        ],
        out_specs=pl.BlockSpec((BN2, 2 * D), lambda i: (i, 0)),
        out_shape=jax.ShapeDtypeStruct((N_TOK // 2, 2 * D), jnp.float32),
        compiler_params=pltpu.CompilerParams(
            dimension_semantics=("arbitrary",)),
    )(x2, w2, p22, phi)
    return out2.reshape(N_TOK, D)


# speedup vs baseline: 3.4558x; 3.4558x over previous
import jax
import jax.numpy as jnp
from jax import lax
from jax.experimental import pallas as pl
from jax.experimental.pallas import tpu as pltpu

N_TOK = 65536
C = 1024
D = 64
BT = 2048
NB = N_TOK // BT


def _codebook_kernel(mp_ref, eps_ref, waug_ref, phit_ref):
    mp = mp_ref[...]
    eps = eps_ref[...]
    phi = jnp.dot(mp, eps, preferred_element_type=jnp.float32)
    p2 = jnp.sum(phi * phi, axis=1, keepdims=True)
    waug_ref[...] = jnp.concatenate([-2.0 * phi, p2], axis=1)
    phit_ref[...] = lax.dot_general(eps, mp, (((0,), (1,)), ((), ())),
                                    preferred_element_type=jnp.float32)


def _vq_kernel(xt_ref, waug_ref, phit_ref, out_ref):
    xt = xt_ref[...]
    xaug = jnp.concatenate([xt, jnp.ones((1, BT), jnp.float32)], axis=0)
    d2t = jnp.dot(waug_ref[...], xaug, preferred_element_type=jnp.float32)
    idx = jnp.argmin(d2t, axis=0, keepdims=True).astype(jnp.int32)
    ids = lax.broadcasted_iota(jnp.int32, d2t.shape, 0)
    onehot = jnp.where(ids == idx, 1.0, 0.0)
    out_ref[...] = jnp.dot(phit_ref[...], onehot,
                           preferred_element_type=jnp.float32)


def kernel(X, matrix_parents, epsilon):
    waug, phit = pl.pallas_call(
        _codebook_kernel,
        out_shape=[
            jax.ShapeDtypeStruct((C, D + 1), jnp.float32),
            jax.ShapeDtypeStruct((D, C), jnp.float32),
        ],
    )(matrix_parents, epsilon)

    out_t = pl.pallas_call(
        _vq_kernel,
        grid=(NB,),
        in_specs=[
            pl.BlockSpec((D, BT), lambda i: (0, i)),
            pl.BlockSpec((C, D + 1), lambda i: (0, 0)),
            pl.BlockSpec((D, C), lambda i: (0, 0)),
        ],
        out_specs=pl.BlockSpec((D, BT), lambda i: (0, i)),
        out_shape=jax.ShapeDtypeStruct((D, N_TOK), jnp.float32),
        compiler_params=pltpu.CompilerParams(
            dimension_semantics=("arbitrary",)),
    )(X.T, waug, phit)
    return out_t.T
